# Initial kernel scaffold; baseline (speedup 1.0000x reference)
import jax
import jax.numpy as jnp
from jax.experimental import pallas as pl


def kernel(imu_signal, W_mag, b_mag, W_phase, b_phase, codebook):
    raise NotImplementedError("write your pallas kernel here")


# speedup vs baseline: 1.1684x; 1.1684x over previous
import functools

import jax
import jax.numpy as jnp
import numpy as np
from jax import lax
from jax.experimental import pallas as pl
from jax.experimental.pallas import tpu as pltpu
from jax.experimental.pallas import tpu_sc as plsc

NUM_CODES = 8192
CODE_DIM = 256
HALF = CODE_DIM // 2
B, C, T = 16, 9, 1024

CODE_TILE = 1024
NJ = NUM_CODES // CODE_TILE


def _hilbert_matrix_t(t: int) -> np.ndarray:
    h = np.zeros((t,), dtype=np.float64)
    h[0] = 1.0
    h[1 : t // 2] = 2.0
    if t % 2 == 0:
        h[t // 2] = 1.0
    f = np.fft.fft(np.eye(t))
    m = np.fft.ifft(h[:, None] * f, axis=0).imag
    return np.ascontiguousarray(m.T).astype(np.float32)


_HILB_MT = _hilbert_matrix_t(T)


def _vq_body(x_ref, mt_ref, wm_ref, bm_ref, wp_ref, bp_ref, cb_ref,
             idx_out, ph_out, feat_scr, ph_scr, f2_scr, best_d2, best_idx):
    j = pl.program_id(1)

    @pl.when(j == 0)
    def _features():
        x = x_ref[0]
        hx = jnp.dot(x, mt_ref[...], precision=lax.Precision.HIGHEST,
                     preferred_element_type=jnp.float32)
        ph = jnp.arctan2(hx, x)
        phases = jnp.mean(ph, axis=0, keepdims=True)
        ph_scr[...] = phases
        mag_t = jnp.dot(wm_ref[...], x, preferred_element_type=jnp.float32)
        mag_t = mag_t + bm_ref[...]
        combined = jnp.concatenate(
            [x[:7, :], jnp.cos(phases), jnp.sin(phases)], axis=0)
        ph_t = jnp.dot(wp_ref[...], combined, preferred_element_type=jnp.float32)
        ph_t = ph_t + bp_ref[...]
        feat = jnp.concatenate([mag_t, ph_t], axis=0)
        feat_scr[...] = feat
        f2_scr[...] = jnp.sum(feat * feat, axis=0, keepdims=True)

    cb = cb_ref[...]
    cross = jnp.dot(cb, feat_scr[...], preferred_element_type=jnp.float32)
    c2 = jnp.sum(cb * cb, axis=1, keepdims=True)
    d2 = (f2_scr[...] + c2) - 2.0 * cross
    score = jnp.sqrt(jnp.maximum(d2, 0.0))
    loc_min = jnp.min(score, axis=0, keepdims=True)
    iot = lax.broadcasted_iota(jnp.int32, score.shape, 0)
    loc_arg = jnp.min(jnp.where(score == loc_min, iot, jnp.int32(2**30)),
                      axis=0, keepdims=True) + j * CODE_TILE

    @pl.when(j == 0)
    def _init():
        best_d2[...] = loc_min
        best_idx[...] = loc_arg

    @pl.when(j > 0)
    def _update():
        bd = best_d2[...]
        upd = loc_min < bd
        best_d2[...] = jnp.where(upd, loc_min, bd)
        best_idx[...] = jnp.where(upd, loc_arg, best_idx[...])

    @pl.when(j == NJ - 1)
    def _emit():
        idx_out[0] = best_idx[...]
        ph_out[0] = ph_scr[...]


def _vq_tc(imu_signal, mt, w_mag, b_mag_col, w_phase, b_phase_col, codebook):
    grid = (B, NJ)
    return pl.pallas_call(
        _vq_body,
        grid=grid,
        in_specs=[
            pl.BlockSpec((1, C, T), lambda i, j: (i, 0, 0)),
            pl.BlockSpec((T, T), lambda i, j: (0, 0)),
            pl.BlockSpec((HALF, C), lambda i, j: (0, 0)),
            pl.BlockSpec((HALF, 1), lambda i, j: (0, 0)),
            pl.BlockSpec((HALF, C), lambda i, j: (0, 0)),
            pl.BlockSpec((HALF, 1), lambda i, j: (0, 0)),
            pl.BlockSpec((CODE_TILE, CODE_DIM), lambda i, j: (j, 0)),
        ],
        out_specs=[
            pl.BlockSpec((1, 1, T), lambda i, j: (i, 0, 0)),
            pl.BlockSpec((1, 1, T), lambda i, j: (i, 0, 0)),
        ],
        out_shape=[
            jax.ShapeDtypeStruct((B, 1, T), jnp.int32),
            jax.ShapeDtypeStruct((B, 1, T), jnp.float32),
        ],
        scratch_shapes=[
            pltpu.VMEM((CODE_DIM, T), jnp.float32),
            pltpu.VMEM((1, T), jnp.float32),
            pltpu.VMEM((1, T), jnp.float32),
            pltpu.VMEM((1, T), jnp.float32),
            pltpu.VMEM((1, T), jnp.int32),
        ],
        compiler_params=pltpu.CompilerParams(
            dimension_semantics=("arbitrary", "arbitrary")),
    )(imu_signal, mt, w_mag, b_mag_col, w_phase, b_phase_col, codebook)


def _sc_gather(codebook, idx_flat):
    info = plsc.get_sparse_core_info()
    nw = info.num_cores * info.num_subcores
    rows_per_w = (B * T) // nw
    chunk = 128
    nchunks = rows_per_w // chunk

    @functools.partial(
        pl.kernel,
        mesh=plsc.VectorSubcoreMesh(core_axis_name="c", subcore_axis_name="s"),
        out_type=jax.ShapeDtypeStruct((B * T, CODE_DIM), jnp.float32),
        scratch_types=[
            pltpu.VMEM((chunk,), jnp.int32),
            pltpu.VMEM((chunk, CODE_DIM), jnp.float32),
            pltpu.SemaphoreType.DMA,
        ],
    )
    def gather_k(cb_hbm, idx_hbm, out_hbm, idx_v, rows_v, sem):
        wid = lax.axis_index("s") * info.num_cores + lax.axis_index("c")
        for t in range(nchunks):
            base = wid * rows_per_w + t * chunk
            pltpu.sync_copy(idx_hbm.at[pl.ds(base, chunk)], idx_v)
            pltpu.async_copy(cb_hbm.at[idx_v], rows_v, sem).wait()
            pltpu.sync_copy(rows_v, out_hbm.at[pl.ds(base, chunk)])

    return gather_k(codebook, idx_flat)


def kernel(imu_signal, W_mag, b_mag, W_phase, b_phase, codebook):
    mt = jnp.asarray(_HILB_MT)
    idx3, ph3 = _vq_tc(imu_signal, mt, W_mag, b_mag.reshape(HALF, 1),
                       W_phase, b_phase.reshape(HALF, 1), codebook)
    indices = idx3.reshape(B, T)
    phases = ph3.reshape(B, T)
    quantized = _sc_gather(codebook, idx3.reshape(B * T))
    return quantized.reshape(B, T, CODE_DIM), indices, phases


# speedup vs baseline: 1.2533x; 1.0726x over previous
import functools

import jax
import jax.numpy as jnp
import numpy as np
from jax import lax
from jax.experimental import pallas as pl
from jax.experimental.pallas import tpu as pltpu
from jax.experimental.pallas import tpu_sc as plsc

NUM_CODES = 8192
CODE_DIM = 256
HALF = CODE_DIM // 2
B, C, T = 16, 9, 1024

CODE_TILE = 1024
NJ = NUM_CODES // CODE_TILE


def _hilbert_matrix_t(t: int) -> np.ndarray:
    h = np.zeros((t,), dtype=np.float64)
    h[0] = 1.0
    h[1 : t // 2] = 2.0
    if t % 2 == 0:
        h[t // 2] = 1.0
    f = np.fft.fft(np.eye(t))
    m = np.fft.ifft(h[:, None] * f, axis=0).imag
    return np.ascontiguousarray(m.T).astype(np.float32)


_HILB_MT = _hilbert_matrix_t(T)


def _feat_body(x_ref, mt_ref, wm_ref, bm_ref, wp_ref, bp_ref,
               feat_out, f2_out, ph_out):
    x = x_ref[0]
    hx = jnp.dot(x, mt_ref[...], precision=lax.Precision.HIGHEST,
                 preferred_element_type=jnp.float32)
    ph = jnp.arctan2(hx, x)
    phases = jnp.mean(ph, axis=0, keepdims=True)
    ph_out[0] = phases
    mag_t = jnp.dot(wm_ref[...], x, preferred_element_type=jnp.float32)
    mag_t = mag_t + bm_ref[...]
    combined = jnp.concatenate(
        [x[:7, :], jnp.cos(phases), jnp.sin(phases)], axis=0)
    ph_t = jnp.dot(wp_ref[...], combined, preferred_element_type=jnp.float32)
    ph_t = ph_t + bp_ref[...]
    feat = jnp.concatenate([mag_t, ph_t], axis=0)
    feat_out[0] = feat
    f2_out[0] = jnp.sum(feat * feat, axis=0, keepdims=True)


def _features_tc(imu_signal, mt, w_mag, b_mag_col, w_phase, b_phase_col):
    return pl.pallas_call(
        _feat_body,
        grid=(B,),
        in_specs=[
            pl.BlockSpec((1, C, T), lambda i: (i, 0, 0)),
            pl.BlockSpec((T, T), lambda i: (0, 0)),
            pl.BlockSpec((HALF, C), lambda i: (0, 0)),
            pl.BlockSpec((HALF, 1), lambda i: (0, 0)),
            pl.BlockSpec((HALF, C), lambda i: (0, 0)),
            pl.BlockSpec((HALF, 1), lambda i: (0, 0)),
        ],
        out_specs=[
            pl.BlockSpec((1, CODE_DIM, T), lambda i: (i, 0, 0)),
            pl.BlockSpec((1, 1, T), lambda i: (i, 0, 0)),
            pl.BlockSpec((1, 1, T), lambda i: (i, 0, 0)),
        ],
        out_shape=[
            jax.ShapeDtypeStruct((B, CODE_DIM, T), jnp.float32),
            jax.ShapeDtypeStruct((B, 1, T), jnp.float32),
            jax.ShapeDtypeStruct((B, 1, T), jnp.float32),
        ],
        compiler_params=pltpu.CompilerParams(
            dimension_semantics=("parallel",)),
    )(imu_signal, mt, w_mag, b_mag_col, w_phase, b_phase_col)


def _vq_body(feat_ref, f2_ref, cb_ref, idx_out, best_d2, best_idx):
    j = pl.program_id(1)
    cb = cb_ref[...]
    cross = jnp.dot(cb, feat_ref[0], preferred_element_type=jnp.float32)
    c2 = jnp.sum(cb * cb, axis=1, keepdims=True)
    d2 = (f2_ref[0] + c2) - 2.0 * cross
    score = jnp.sqrt(jnp.maximum(d2, 0.0))
    loc_min_d2 = jnp.min(d2, axis=0, keepdims=True)
    loc_min = jnp.sqrt(jnp.maximum(loc_min_d2, 0.0))
    loc_arg = (jnp.argmin(score, axis=0, keepdims=True).astype(jnp.int32)
               + j * CODE_TILE)

    @pl.when(j == 0)
    def _init():
        best_d2[...] = loc_min
        best_idx[...] = loc_arg

    @pl.when(j > 0)
    def _update():
        bd = best_d2[...]
        upd = loc_min < bd
        best_d2[...] = jnp.where(upd, loc_min, bd)
        best_idx[...] = jnp.where(upd, loc_arg, best_idx[...])

    @pl.when(j == NJ - 1)
    def _emit():
        idx_out[0] = best_idx[...]


def _vq_tc(feats, f2, codebook):
    return pl.pallas_call(
        _vq_body,
        grid=(B, NJ),
        in_specs=[
            pl.BlockSpec((1, CODE_DIM, T), lambda i, j: (i, 0, 0)),
            pl.BlockSpec((1, 1, T), lambda i, j: (i, 0, 0)),
            pl.BlockSpec((CODE_TILE, CODE_DIM), lambda i, j: (j, 0)),
        ],
        out_specs=pl.BlockSpec((1, 1, T), lambda i, j: (i, 0, 0)),
        out_shape=jax.ShapeDtypeStruct((B, 1, T), jnp.int32),
        scratch_shapes=[
            pltpu.VMEM((1, T), jnp.float32),
            pltpu.VMEM((1, T), jnp.int32),
        ],
        compiler_params=pltpu.CompilerParams(
            dimension_semantics=("arbitrary", "arbitrary")),
    )(feats, f2, codebook)


def _sc_gather(codebook, idx_flat):
    info = plsc.get_sparse_core_info()
    nw = info.num_cores * info.num_subcores
    rows_per_w = (B * T) // nw
    chunk = 128
    nchunks = rows_per_w // chunk

    @functools.partial(
        pl.kernel,
        mesh=plsc.VectorSubcoreMesh(core_axis_name="c", subcore_axis_name="s"),
        out_type=jax.ShapeDtypeStruct((B * T, CODE_DIM), jnp.float32),
        scratch_types=[
            pltpu.VMEM((chunk,), jnp.int32),
            pltpu.VMEM((chunk, CODE_DIM), jnp.float32),
            pltpu.SemaphoreType.DMA,
        ],
    )
    def gather_k(cb_hbm, idx_hbm, out_hbm, idx_v, rows_v, sem):
        wid = lax.axis_index("s") * info.num_cores + lax.axis_index("c")
        for t in range(nchunks):
            base = wid * rows_per_w + t * chunk
            pltpu.sync_copy(idx_hbm.at[pl.ds(base, chunk)], idx_v)
            pltpu.async_copy(cb_hbm.at[idx_v], rows_v, sem).wait()
            pltpu.sync_copy(rows_v, out_hbm.at[pl.ds(base, chunk)])

    return gather_k(codebook, idx_flat)


def kernel(imu_signal, W_mag, b_mag, W_phase, b_phase, codebook):
    mt = jnp.asarray(_HILB_MT)
    feats, f2, ph3 = _features_tc(imu_signal, mt, W_mag,
                                  b_mag.reshape(HALF, 1), W_phase,
                                  b_phase.reshape(HALF, 1))
    idx3 = _vq_tc(feats, f2, codebook)
    indices = idx3.reshape(B, T)
    phases = ph3.reshape(B, T)
    quantized = _sc_gather(codebook, idx3.reshape(B * T))
    return quantized.reshape(B, T, CODE_DIM), indices, phases


# speedup vs baseline: 1.2544x; 1.0009x over previous
import functools

import jax
import jax.numpy as jnp
import numpy as np
from jax import lax
from jax.experimental import pallas as pl
from jax.experimental.pallas import tpu as pltpu
from jax.experimental.pallas import tpu_sc as plsc

NUM_CODES = 8192
CODE_DIM = 256
HALF = CODE_DIM // 2
B, C, T = 16, 9, 1024

CODE_TILE = 1024
NJ = NUM_CODES // CODE_TILE


def _hilbert_matrix_t(t: int) -> np.ndarray:
    h = np.zeros((t,), dtype=np.float64)
    h[0] = 1.0
    h[1 : t // 2] = 2.0
    if t % 2 == 0:
        h[t // 2] = 1.0
    f = np.fft.fft(np.eye(t))
    m = np.fft.ifft(h[:, None] * f, axis=0).imag
    return np.ascontiguousarray(m.T).astype(np.float32)


_HILB_MT = _hilbert_matrix_t(T)


def _feat_body(x_ref, mt_ref, wm_ref, bm_ref, wp_ref, bp_ref,
               feat_out, f2_out, ph_out):
    x = x_ref[0]
    hx = jnp.dot(x, mt_ref[...], precision=lax.Precision.HIGHEST,
                 preferred_element_type=jnp.float32)
    ph = jnp.arctan2(hx, x)
    phases = jnp.mean(ph, axis=0, keepdims=True)
    ph_out[0] = phases
    mag_t = jnp.dot(wm_ref[...], x, preferred_element_type=jnp.float32)
    mag_t = mag_t + bm_ref[...]
    combined = jnp.concatenate(
        [x[:7, :], jnp.cos(phases), jnp.sin(phases)], axis=0)
    ph_t = jnp.dot(wp_ref[...], combined, preferred_element_type=jnp.float32)
    ph_t = ph_t + bp_ref[...]
    feat = jnp.concatenate([mag_t, ph_t], axis=0)
    feat_out[0] = feat
    f2_out[0] = jnp.sum(feat * feat, axis=0, keepdims=True)


def _features_tc(imu_signal, mt, w_mag, b_mag_col, w_phase, b_phase_col):
    return pl.pallas_call(
        _feat_body,
        grid=(B,),
        in_specs=[
            pl.BlockSpec((1, C, T), lambda i: (i, 0, 0)),
            pl.BlockSpec((T, T), lambda i: (0, 0)),
            pl.BlockSpec((HALF, C), lambda i: (0, 0)),
            pl.BlockSpec((HALF, 1), lambda i: (0, 0)),
            pl.BlockSpec((HALF, C), lambda i: (0, 0)),
            pl.BlockSpec((HALF, 1), lambda i: (0, 0)),
        ],
        out_specs=[
            pl.BlockSpec((1, CODE_DIM, T), lambda i: (i, 0, 0)),
            pl.BlockSpec((1, 1, T), lambda i: (i, 0, 0)),
            pl.BlockSpec((1, 1, T), lambda i: (i, 0, 0)),
        ],
        out_shape=[
            jax.ShapeDtypeStruct((B, CODE_DIM, T), jnp.float32),
            jax.ShapeDtypeStruct((B, 1, T), jnp.float32),
            jax.ShapeDtypeStruct((B, 1, T), jnp.float32),
        ],
        compiler_params=pltpu.CompilerParams(
            dimension_semantics=("parallel",)),
    )(imu_signal, mt, w_mag, b_mag_col, w_phase, b_phase_col)


def _vq_body(feat_ref, f2_ref, cb_ref, idx_out, best_d2, best_idx):
    j = pl.program_id(1)
    cb = cb_ref[...]
    cross = jnp.dot(cb, feat_ref[0], preferred_element_type=jnp.float32)
    c2 = jnp.sum(cb * cb, axis=1, keepdims=True)
    d2 = (f2_ref[0] + c2) - 2.0 * cross
    score = jnp.sqrt(jnp.maximum(d2, 0.0))
    loc_min_d2 = jnp.min(d2, axis=0, keepdims=True)
    loc_min = jnp.sqrt(jnp.maximum(loc_min_d2, 0.0))
    loc_arg = (jnp.argmin(score, axis=0, keepdims=True).astype(jnp.int32)
               + j * CODE_TILE)

    @pl.when(j == 0)
    def _init():
        best_d2[...] = loc_min
        best_idx[...] = loc_arg

    @pl.when(j > 0)
    def _update():
        bd = best_d2[...]
        upd = loc_min < bd
        best_d2[...] = jnp.where(upd, loc_min, bd)
        best_idx[...] = jnp.where(upd, loc_arg, best_idx[...])

    @pl.when(j == NJ - 1)
    def _emit():
        idx_out[0] = best_idx[...]


def _vq_tc(feats, f2, codebook):
    return pl.pallas_call(
        _vq_body,
        grid=(B, NJ),
        in_specs=[
            pl.BlockSpec((1, CODE_DIM, T), lambda i, j: (i, 0, 0)),
            pl.BlockSpec((1, 1, T), lambda i, j: (i, 0, 0)),
            pl.BlockSpec((CODE_TILE, CODE_DIM), lambda i, j: (j, 0)),
        ],
        out_specs=pl.BlockSpec((1, 1, T), lambda i, j: (i, 0, 0)),
        out_shape=jax.ShapeDtypeStruct((B, 1, T), jnp.int32),
        scratch_shapes=[
            pltpu.VMEM((1, T), jnp.float32),
            pltpu.VMEM((1, T), jnp.int32),
        ],
        compiler_params=pltpu.CompilerParams(
            dimension_semantics=("arbitrary", "arbitrary")),
    )(feats, f2, codebook)


def _sc_gather(codebook, idx_flat):
    info = plsc.get_sparse_core_info()
    nw = info.num_cores * info.num_subcores
    rows_per_w = (B * T) // nw
    chunk = 64
    nchunks = rows_per_w // chunk
    nbuf = 4

    @functools.partial(
        pl.kernel,
        mesh=plsc.VectorSubcoreMesh(core_axis_name="c", subcore_axis_name="s"),
        out_type=jax.ShapeDtypeStruct((B * T, CODE_DIM), jnp.float32),
        scratch_types=[
            pltpu.VMEM((rows_per_w,), jnp.int32),
        ]
        + [pltpu.VMEM((chunk, CODE_DIM), jnp.float32) for _ in range(nbuf)]
        + [pltpu.SemaphoreType.DMA for _ in range(nbuf)],
    )
    def gather_k(cb_hbm, idx_hbm, out_hbm, idx_v, *bufs_sems):
        bufs, sems = bufs_sems[:nbuf], bufs_sems[nbuf:]
        wid = lax.axis_index("s") * info.num_cores + lax.axis_index("c")
        base = wid * rows_per_w
        pltpu.sync_copy(idx_hbm.at[pl.ds(base, rows_per_w)], idx_v)

        def fire(t):
            return pltpu.async_copy(
                cb_hbm.at[idx_v.at[pl.ds(t * chunk, chunk)]],
                bufs[t % nbuf], sems[t % nbuf])

        inflight = [fire(t) for t in range(nbuf)]
        for t in range(nchunks):
            inflight[t % nbuf].wait()
            pltpu.sync_copy(bufs[t % nbuf],
                            out_hbm.at[pl.ds(base + t * chunk, chunk)])
            if t + nbuf < nchunks:
                inflight[t % nbuf] = fire(t + nbuf)

    return gather_k(codebook, idx_flat)


def kernel(imu_signal, W_mag, b_mag, W_phase, b_phase, codebook):
    mt = jnp.asarray(_HILB_MT)
    feats, f2, ph3 = _features_tc(imu_signal, mt, W_mag,
                                  b_mag.reshape(HALF, 1), W_phase,
                                  b_phase.reshape(HALF, 1))
    idx3 = _vq_tc(feats, f2, codebook)
    indices = idx3.reshape(B, T)
    phases = ph3.reshape(B, T)
    quantized = _sc_gather(codebook, idx3.reshape(B * T))
    return quantized.reshape(B, T, CODE_DIM), indices, phases


# speedup vs baseline: 1.3138x; 1.0474x over previous
import functools

import jax
import jax.numpy as jnp
import numpy as np
from jax import lax
from jax.experimental import pallas as pl
from jax.experimental.pallas import tpu as pltpu
from jax.experimental.pallas import tpu_sc as plsc

NUM_CODES = 8192
CODE_DIM = 256
HALF = CODE_DIM // 2
B, C, T = 16, 9, 1024

CODE_TILE = 2048
NJ = NUM_CODES // CODE_TILE


def _hilbert_matrix_t(t: int) -> np.ndarray:
    h = np.zeros((t,), dtype=np.float64)
    h[0] = 1.0
    h[1 : t // 2] = 2.0
    if t % 2 == 0:
        h[t // 2] = 1.0
    f = np.fft.fft(np.eye(t))
    m = np.fft.ifft(h[:, None] * f, axis=0).imag
    return np.ascontiguousarray(m.T).astype(np.float32)


_HILB_MT = _hilbert_matrix_t(T)


def _feat_body(x_ref, mt_ref, wm_ref, bm_ref, wp_ref, bp_ref,
               feat_out, f2_out, ph_out):
    x = x_ref[0]
    hx = jnp.dot(x, mt_ref[...], precision=lax.Precision.HIGHEST,
                 preferred_element_type=jnp.float32)
    ph = jnp.arctan2(hx, x)
    phases = jnp.mean(ph, axis=0, keepdims=True)
    ph_out[0] = phases
    mag_t = jnp.dot(wm_ref[...], x, preferred_element_type=jnp.float32)
    mag_t = mag_t + bm_ref[...]
    combined = jnp.concatenate(
        [x[:7, :], jnp.cos(phases), jnp.sin(phases)], axis=0)
    ph_t = jnp.dot(wp_ref[...], combined, preferred_element_type=jnp.float32)
    ph_t = ph_t + bp_ref[...]
    feat = jnp.concatenate([mag_t, ph_t], axis=0)
    feat_out[0] = feat
    f2_out[0] = jnp.sum(feat * feat, axis=0, keepdims=True)


def _features_tc(imu_signal, mt, w_mag, b_mag_col, w_phase, b_phase_col):
    return pl.pallas_call(
        _feat_body,
        grid=(B,),
        in_specs=[
            pl.BlockSpec((1, C, T), lambda i: (i, 0, 0)),
            pl.BlockSpec((T, T), lambda i: (0, 0)),
            pl.BlockSpec((HALF, C), lambda i: (0, 0)),
            pl.BlockSpec((HALF, 1), lambda i: (0, 0)),
            pl.BlockSpec((HALF, C), lambda i: (0, 0)),
            pl.BlockSpec((HALF, 1), lambda i: (0, 0)),
        ],
        out_specs=[
            pl.BlockSpec((1, CODE_DIM, T), lambda i: (i, 0, 0)),
            pl.BlockSpec((1, 1, T), lambda i: (i, 0, 0)),
            pl.BlockSpec((1, 1, T), lambda i: (i, 0, 0)),
        ],
        out_shape=[
            jax.ShapeDtypeStruct((B, CODE_DIM, T), jnp.float32),
            jax.ShapeDtypeStruct((B, 1, T), jnp.float32),
            jax.ShapeDtypeStruct((B, 1, T), jnp.float32),
        ],
        compiler_params=pltpu.CompilerParams(
            dimension_semantics=("parallel",)),
    )(imu_signal, mt, w_mag, b_mag_col, w_phase, b_phase_col)


def _vq_body(feat_ref, f2_ref, cb_ref, idx_out, best_d2, best_idx):
    j = pl.program_id(1)
    cb = cb_ref[...]
    cross = jnp.dot(cb, feat_ref[0], preferred_element_type=jnp.float32)
    c2 = jnp.sum(cb * cb, axis=1, keepdims=True)
    d2 = (f2_ref[0] + c2) - 2.0 * cross
    score = jnp.sqrt(jnp.maximum(d2, 0.0))
    loc_min_d2 = jnp.min(d2, axis=0, keepdims=True)
    loc_min = jnp.sqrt(jnp.maximum(loc_min_d2, 0.0))
    loc_arg = (jnp.argmin(score, axis=0, keepdims=True).astype(jnp.int32)
               + j * CODE_TILE)

    @pl.when(j == 0)
    def _init():
        best_d2[...] = loc_min
        best_idx[...] = loc_arg

    @pl.when(j > 0)
    def _update():
        bd = best_d2[...]
        upd = loc_min < bd
        best_d2[...] = jnp.where(upd, loc_min, bd)
        best_idx[...] = jnp.where(upd, loc_arg, best_idx[...])

    @pl.when(j == NJ - 1)
    def _emit():
        idx_out[0] = best_idx[...]


def _vq_tc(feats, f2, codebook):
    return pl.pallas_call(
        _vq_body,
        grid=(B, NJ),
        in_specs=[
            pl.BlockSpec((1, CODE_DIM, T), lambda i, j: (i, 0, 0)),
            pl.BlockSpec((1, 1, T), lambda i, j: (i, 0, 0)),
            pl.BlockSpec((CODE_TILE, CODE_DIM), lambda i, j: (j, 0)),
        ],
        out_specs=pl.BlockSpec((1, 1, T), lambda i, j: (i, 0, 0)),
        out_shape=jax.ShapeDtypeStruct((B, 1, T), jnp.int32),
        scratch_shapes=[
            pltpu.VMEM((1, T), jnp.float32),
            pltpu.VMEM((1, T), jnp.int32),
        ],
        compiler_params=pltpu.CompilerParams(
            dimension_semantics=("arbitrary", "arbitrary")),
    )(feats, f2, codebook)


def _sc_gather(codebook, idx_flat):
    info = plsc.get_sparse_core_info()
    nw = info.num_cores * info.num_subcores
    rows_per_w = (B * T) // nw
    chunk = 64
    nchunks = rows_per_w // chunk
    nbuf = 4

    @functools.partial(
        pl.kernel,
        mesh=plsc.VectorSubcoreMesh(core_axis_name="c", subcore_axis_name="s"),
        out_type=jax.ShapeDtypeStruct((B * T, CODE_DIM), jnp.float32),
        scratch_types=[
            pltpu.VMEM((rows_per_w,), jnp.int32),
        ]
        + [pltpu.VMEM((chunk, CODE_DIM), jnp.float32) for _ in range(nbuf)]
        + [pltpu.SemaphoreType.DMA for _ in range(nbuf)],
    )
    def gather_k(cb_hbm, idx_hbm, out_hbm, idx_v, *bufs_sems):
        bufs, sems = bufs_sems[:nbuf], bufs_sems[nbuf:]
        wid = lax.axis_index("s") * info.num_cores + lax.axis_index("c")
        base = wid * rows_per_w
        pltpu.sync_copy(idx_hbm.at[pl.ds(base, rows_per_w)], idx_v)

        def fire(t):
            return pltpu.async_copy(
                cb_hbm.at[idx_v.at[pl.ds(t * chunk, chunk)]],
                bufs[t % nbuf], sems[t % nbuf])

        inflight = [fire(t) for t in range(nbuf)]
        for t in range(nchunks):
            inflight[t % nbuf].wait()
            pltpu.sync_copy(bufs[t % nbuf],
                            out_hbm.at[pl.ds(base + t * chunk, chunk)])
            if t + nbuf < nchunks:
                inflight[t % nbuf] = fire(t + nbuf)

    return gather_k(codebook, idx_flat)


def kernel(imu_signal, W_mag, b_mag, W_phase, b_phase, codebook):
    mt = jnp.asarray(_HILB_MT)
    feats, f2, ph3 = _features_tc(imu_signal, mt, W_mag,
                                  b_mag.reshape(HALF, 1), W_phase,
                                  b_phase.reshape(HALF, 1))
    idx3 = _vq_tc(feats, f2, codebook)
    indices = idx3.reshape(B, T)
    phases = ph3.reshape(B, T)
    quantized = _sc_gather(codebook, idx3.reshape(B * T))
    return quantized.reshape(B, T, CODE_DIM), indices, phases
